# baseline (speedup 1.0000x reference)
import functools
import math

import jax
import jax.numpy as jnp
from jax import lax
from jax.experimental import pallas as pl
from jax.experimental.pallas import tpu as pltpu
from jax.experimental.pallas import tpu_sc as plsc

_NW = 32

_EC = 2000
_AC = 784


def _sc_gathers(a, e, atom_table, edge_table):
    n_nodes = a.shape[0]
    n_edges = e.shape[0]
    adim = atom_table.shape[1]
    edim = edge_table.shape[1]
    e_per_w = n_edges // _NW
    e_chunks = e_per_w // _EC
    a_chunks_per_w = (n_nodes + _AC * _NW - 1) // (_AC * _NW)
    mesh = plsc.VectorSubcoreMesh(core_axis_name="c", subcore_axis_name="s")

    @functools.partial(
        pl.kernel,
        out_type=(
            jax.ShapeDtypeStruct((n_nodes, adim), jnp.float32),
            jax.ShapeDtypeStruct((n_edges, edim), jnp.float32),
        ),
        mesh=mesh,
        compiler_params=pltpu.CompilerParams(use_tc_tiling_on_sc=False),
        scratch_types=[
            pltpu.VMEM((_EC,), jnp.int32),
            pltpu.VMEM((_EC, edim), jnp.float32),
            pltpu.VMEM((_AC,), jnp.int32),
            pltpu.VMEM((_AC, adim), jnp.float32),
            pltpu.SemaphoreType.DMA,
        ],
    )
    def k(a_hbm, e_hbm, at_hbm, et_hbm, a_out, e_out,
          eidx_v, erows_v, aidx_v, arows_v, sem):
        wid = lax.axis_index("s") * 2 + lax.axis_index("c")

        def ebody(i, carry):
            base = wid * e_per_w + i * _EC
            pltpu.sync_copy(e_hbm.at[pl.ds(base, _EC)], eidx_v)
            pltpu.async_copy(et_hbm.at[eidx_v], erows_v, sem).wait()
            pltpu.sync_copy(erows_v, e_out.at[pl.ds(base, _EC)])
            return carry

        lax.fori_loop(0, e_chunks, ebody, 0)

        def abody(j, carry):
            chunk = wid * a_chunks_per_w + j
            base = jnp.minimum(chunk * _AC, n_nodes - _AC)
            pltpu.sync_copy(a_hbm.at[pl.ds(base, _AC)], aidx_v)
            pltpu.async_copy(at_hbm.at[aidx_v], arows_v, sem).wait()
            pltpu.sync_copy(arows_v, a_out.at[pl.ds(base, _AC)])
            return carry

        lax.fori_loop(0, a_chunks_per_w, abody, 0)

    return k(a, e, atom_table, edge_table)


def _tc_cond(t_col, batch2d, node_count_table, W_t, b_row):
    num_graphs = t_col.shape[0]
    ncv, ncd = node_count_table.shape
    tdim = W_t.shape[0]
    half = tdim // 2
    nblk, blk = batch2d.shape

    def body(t_ref, batch_ref, nct_ref, wt_ref, bt_ref, out_ref):
        gid_col = lax.broadcasted_iota(jnp.int32, (num_graphs, 1), 0)

        def cbody(i, acc):
            row = batch_ref[pl.ds(i, 1), :]
            eq = (row == gid_col).astype(jnp.float32)
            return acc + jnp.sum(eq, axis=1, keepdims=True)

        counts = lax.fori_loop(
            0, nblk, cbody, jnp.zeros((num_graphs, 1), jnp.float32))
        n_idx = jnp.clip(counts.astype(jnp.int32), 0, ncv - 1)
        onehot = (n_idx == lax.broadcasted_iota(
            jnp.int32, (1, ncv), 1)).astype(jnp.float32)
        n_embed = jnp.dot(onehot, nct_ref[...],
                          preferred_element_type=jnp.float32)

        freqs = jnp.exp(
            (-math.log(10000.0) / half)
            * lax.broadcasted_iota(jnp.int32, (1, half), 1).astype(jnp.float32))
        args = t_ref[...] * freqs
        temb = jnp.concatenate([jnp.sin(args), jnp.cos(args)], axis=-1)
        t_embed = jnp.dot(temb, wt_ref[...],
                          preferred_element_type=jnp.float32) + bt_ref[...]

        out_ref[:, :tdim] = t_embed
        out_ref[:, tdim:] = n_embed

    return pl.pallas_call(
        body,
        out_shape=jax.ShapeDtypeStruct((num_graphs, tdim + ncd), jnp.float32),
    )(t_col, batch2d, node_count_table, W_t, b_row)


def kernel(a, e, edge_index, t, batch, atom_table, edge_table,
           node_count_table, W_t, b_t):
    del edge_index
    a_embed, e_embed = _sc_gathers(a, e, atom_table, edge_table)
    cond = _tc_cond(
        t.reshape(-1, 1),
        batch.reshape(25, -1),
        node_count_table,
        W_t,
        b_t.reshape(1, -1),
    )
    return a_embed, e_embed, cond


# speedup vs baseline: 3.1107x; 3.1107x over previous
import functools
import math

import jax
import jax.numpy as jnp
from jax import lax
from jax.experimental import pallas as pl
from jax.experimental.pallas import tpu as pltpu
from jax.experimental.pallas import tpu_sc as plsc

_NW = 32
_L = 16

_EC = 1024
_AC = 512


def _expand_chunk(idx_ref, table_ref, rows_ref, nrows, ncols):
    iota = lax.broadcasted_iota(jnp.int32, (_L,), 0)

    def gbody(g, carry):
        ev = idx_ref[pl.ds(g * _L, _L)]
        rows = g * _L + iota
        for c in range(ncols):
            colv = jnp.full((_L,), c, jnp.int32)
            vals = plsc.load_gather(table_ref, [ev, colv])
            plsc.store_scatter(rows_ref, [rows, colv], vals)
        return carry

    lax.fori_loop(0, nrows // _L, gbody, 0)


def _sc_gathers(a, e, atom_table, edge_table):
    n_nodes = a.shape[0]
    n_edges = e.shape[0]
    av, adim = atom_table.shape
    ev_, edim = edge_table.shape
    e_per_w = n_edges // _NW
    e_chunks = (e_per_w + _EC - 1) // _EC
    a_chunks_per_w = (n_nodes + _AC * _NW - 1) // (_AC * _NW)
    mesh = plsc.VectorSubcoreMesh(core_axis_name="c", subcore_axis_name="s")

    @functools.partial(
        pl.kernel,
        out_type=(
            jax.ShapeDtypeStruct((n_nodes, adim), jnp.float32),
            jax.ShapeDtypeStruct((n_edges, edim), jnp.float32),
        ),
        mesh=mesh,
        compiler_params=pltpu.CompilerParams(
            use_tc_tiling_on_sc=False, needs_layout_passes=False),
        scratch_types=[
            pltpu.VMEM((ev_, edim), jnp.float32),
            pltpu.VMEM((av, adim), jnp.float32),
            pltpu.VMEM((2, _EC), jnp.int32),
            pltpu.VMEM((2, _EC, edim), jnp.float32),
            pltpu.VMEM((_AC,), jnp.int32),
            pltpu.VMEM((_AC, adim), jnp.float32),
            pltpu.SemaphoreType.DMA((2,)),
            pltpu.SemaphoreType.DMA((2,)),
            pltpu.SemaphoreType.DMA,
        ],
    )
    def k(a_hbm, e_hbm, at_hbm, et_hbm, a_out, e_out,
          et_v, at_v, eidx_v, erows_v, aidx_v, arows_v,
          sem_in, sem_out, sem_a):
        wid = lax.axis_index("s") * 2 + lax.axis_index("c")
        w_base = wid * e_per_w
        pltpu.sync_copy(et_hbm, et_v)
        pltpu.sync_copy(at_hbm, at_v)

        def ebase(i):
            return w_base + jnp.minimum(i * _EC, e_per_w - _EC)

        pltpu.async_copy(e_hbm.at[pl.ds(w_base, _EC)], eidx_v.at[0],
                         sem_in.at[0])

        def ebody(i, carry):
            buf = lax.rem(i, 2)
            pltpu.make_async_copy(
                e_hbm.at[pl.ds(ebase(i), _EC)], eidx_v.at[buf],
                sem_in.at[buf]).wait()

            @pl.when(i + 1 < e_chunks)
            def _():
                pltpu.async_copy(
                    e_hbm.at[pl.ds(ebase(i + 1), _EC)], eidx_v.at[1 - buf],
                    sem_in.at[1 - buf])

            @pl.when(i >= 2)
            def _():
                pltpu.make_async_copy(
                    erows_v.at[buf],
                    e_out.at[pl.ds(ebase(i - 2), _EC)],
                    sem_out.at[buf]).wait()

            _expand_chunk(eidx_v.at[buf], et_v, erows_v.at[buf], _EC, edim)
            pltpu.async_copy(erows_v.at[buf], e_out.at[pl.ds(ebase(i), _EC)],
                             sem_out.at[buf])
            return carry

        lax.fori_loop(0, e_chunks, ebody, 0)
        for tail in (e_chunks - 2, e_chunks - 1):
            pltpu.make_async_copy(
                erows_v.at[tail % 2],
                e_out.at[pl.ds(ebase(tail), _EC)],
                sem_out.at[tail % 2]).wait()

        def abody(j, carry):
            chunk = wid * a_chunks_per_w + j
            base = jnp.minimum(chunk * _AC, n_nodes - _AC)

            @pl.when(j > 0)
            def _():
                pltpu.make_async_copy(
                    arows_v, a_out.at[pl.ds(base, _AC)], sem_a).wait()

            pltpu.sync_copy(a_hbm.at[pl.ds(base, _AC)], aidx_v)
            _expand_chunk(aidx_v, at_v, arows_v, _AC, adim)
            pltpu.async_copy(arows_v, a_out.at[pl.ds(base, _AC)], sem_a)
            return base

        last_base = lax.fori_loop(0, a_chunks_per_w, abody, 0)
        pltpu.make_async_copy(
            arows_v, a_out.at[pl.ds(last_base, _AC)], sem_a).wait()

    return k(a, e, atom_table, edge_table)


def _tc_cond(t_col, batch2d, node_count_table, W_t, b_row):
    num_graphs = t_col.shape[0]
    ncv, ncd = node_count_table.shape
    tdim = W_t.shape[0]
    half = tdim // 2
    nblk, blk = batch2d.shape

    def body(t_ref, batch_ref, nct_ref, wt_ref, bt_ref, out_ref):
        gid_col = lax.broadcasted_iota(jnp.int32, (num_graphs, 1), 0)

        def cbody(i, acc):
            row = batch_ref[pl.ds(i, 1), :]
            eq = (row == gid_col).astype(jnp.float32)
            return acc + jnp.sum(eq, axis=1, keepdims=True)

        counts = lax.fori_loop(
            0, nblk, cbody, jnp.zeros((num_graphs, 1), jnp.float32))
        n_idx = jnp.clip(counts.astype(jnp.int32), 0, ncv - 1)
        onehot = (n_idx == lax.broadcasted_iota(
            jnp.int32, (1, ncv), 1)).astype(jnp.float32)
        n_embed = jnp.dot(onehot, nct_ref[...],
                          preferred_element_type=jnp.float32,
                          precision=lax.Precision.HIGHEST)

        freqs = jnp.exp(
            (-math.log(10000.0) / half)
            * lax.broadcasted_iota(jnp.int32, (1, half), 1).astype(jnp.float32))
        args = t_ref[...] * freqs
        temb = jnp.concatenate([jnp.sin(args), jnp.cos(args)], axis=-1)
        t_embed = jnp.dot(temb, wt_ref[...],
                          preferred_element_type=jnp.float32) + bt_ref[...]

        out_ref[:, :tdim] = t_embed
        out_ref[:, tdim:] = n_embed

    return pl.pallas_call(
        body,
        out_shape=jax.ShapeDtypeStruct((num_graphs, tdim + ncd), jnp.float32),
    )(t_col, batch2d, node_count_table, W_t, b_row)


def kernel(a, e, edge_index, t, batch, atom_table, edge_table,
           node_count_table, W_t, b_t):
    del edge_index
    a_embed, e_embed = _sc_gathers(a, e, atom_table, edge_table)
    cond = _tc_cond(
        t.reshape(-1, 1),
        batch.reshape(25, -1),
        node_count_table,
        W_t,
        b_t.reshape(1, -1),
    )
    return a_embed, e_embed, cond


# speedup vs baseline: 8.5698x; 2.7549x over previous
import functools
import math

import jax
import jax.numpy as jnp
from jax import lax
from jax.experimental import pallas as pl
from jax.experimental.pallas import tpu as pltpu
from jax.experimental.pallas import tpu_sc as plsc

_NW = 32
_L = 16

_EC = 1024
_AC = 512


def _expand_chunk(idx_ref, table_ref, rows_ref, nrows, ncols):
    iota = lax.broadcasted_iota(jnp.int32, (_L,), 0)
    mask = ncols - 1
    grp = 4

    def gbody(g, carry):
        ev = idx_ref[pl.ds(g * _L, _L)]
        rows = g * _L + iota
        for c0 in range(0, ncols, grp):
            colvs = [lax.bitwise_and(c0 + dc + iota, mask)
                     for dc in range(grp)]
            vals = [plsc.load_gather(table_ref, [ev, colv])
                    for colv in colvs]
            for colv, v in zip(colvs, vals):
                plsc.store_scatter(rows_ref, [rows, colv], v)
        return carry

    lax.fori_loop(0, nrows // _L, gbody, 0)


def _sc_gathers(a, e, atom_table, edge_table):
    n_nodes = a.shape[0]
    n_edges = e.shape[0]
    av, adim = atom_table.shape
    ev_, edim = edge_table.shape
    e_per_w = n_edges // _NW
    e_chunks = (e_per_w + _EC - 1) // _EC
    a_chunks_per_w = (n_nodes + _AC * _NW - 1) // (_AC * _NW)
    mesh = plsc.VectorSubcoreMesh(core_axis_name="c", subcore_axis_name="s")

    @functools.partial(
        pl.kernel,
        out_type=(
            jax.ShapeDtypeStruct((n_nodes, adim), jnp.float32),
            jax.ShapeDtypeStruct((n_edges, edim), jnp.float32),
        ),
        mesh=mesh,
        compiler_params=pltpu.CompilerParams(
            use_tc_tiling_on_sc=False, needs_layout_passes=False),
        scratch_types=[
            pltpu.VMEM((ev_, edim), jnp.float32),
            pltpu.VMEM((av, adim), jnp.float32),
            pltpu.VMEM((2, _EC), jnp.int32),
            pltpu.VMEM((2, _EC, edim), jnp.float32),
            pltpu.VMEM((_AC,), jnp.int32),
            pltpu.VMEM((_AC, adim), jnp.float32),
            pltpu.SemaphoreType.DMA((2,)),
            pltpu.SemaphoreType.DMA((2,)),
            pltpu.SemaphoreType.DMA,
        ],
    )
    def k(a_hbm, e_hbm, at_hbm, et_hbm, a_out, e_out,
          et_v, at_v, eidx_v, erows_v, aidx_v, arows_v,
          sem_in, sem_out, sem_a):
        wid = lax.axis_index("s") * 2 + lax.axis_index("c")
        w_base = wid * e_per_w
        pltpu.sync_copy(et_hbm, et_v)
        pltpu.sync_copy(at_hbm, at_v)

        def ebase(i):
            return w_base + jnp.minimum(i * _EC, e_per_w - _EC)

        pltpu.async_copy(e_hbm.at[pl.ds(w_base, _EC)], eidx_v.at[0],
                         sem_in.at[0])

        def ebody(i, carry):
            buf = lax.rem(i, 2)
            pltpu.make_async_copy(
                e_hbm.at[pl.ds(ebase(i), _EC)], eidx_v.at[buf],
                sem_in.at[buf]).wait()

            @pl.when(i + 1 < e_chunks)
            def _():
                pltpu.async_copy(
                    e_hbm.at[pl.ds(ebase(i + 1), _EC)], eidx_v.at[1 - buf],
                    sem_in.at[1 - buf])

            @pl.when(i >= 2)
            def _():
                pltpu.make_async_copy(
                    erows_v.at[buf],
                    e_out.at[pl.ds(ebase(i - 2), _EC)],
                    sem_out.at[buf]).wait()

            _expand_chunk(eidx_v.at[buf], et_v, erows_v.at[buf], _EC, edim)
            pltpu.async_copy(erows_v.at[buf], e_out.at[pl.ds(ebase(i), _EC)],
                             sem_out.at[buf])
            return carry

        lax.fori_loop(0, e_chunks, ebody, 0)
        for tail in (e_chunks - 2, e_chunks - 1):
            pltpu.make_async_copy(
                erows_v.at[tail % 2],
                e_out.at[pl.ds(ebase(tail), _EC)],
                sem_out.at[tail % 2]).wait()

        def abody(j, carry):
            chunk = wid * a_chunks_per_w + j
            base = jnp.minimum(chunk * _AC, n_nodes - _AC)

            @pl.when(j > 0)
            def _():
                pltpu.make_async_copy(
                    arows_v, a_out.at[pl.ds(base, _AC)], sem_a).wait()

            pltpu.sync_copy(a_hbm.at[pl.ds(base, _AC)], aidx_v)
            _expand_chunk(aidx_v, at_v, arows_v, _AC, adim)
            pltpu.async_copy(arows_v, a_out.at[pl.ds(base, _AC)], sem_a)
            return base

        last_base = lax.fori_loop(0, a_chunks_per_w, abody, 0)
        pltpu.make_async_copy(
            arows_v, a_out.at[pl.ds(last_base, _AC)], sem_a).wait()

    return k(a, e, atom_table, edge_table)


def _tc_cond(t_col, batch2d, node_count_table, W_t, b_row):
    num_graphs = t_col.shape[0]
    ncv, ncd = node_count_table.shape
    tdim = W_t.shape[0]
    half = tdim // 2
    nblk, blk = batch2d.shape

    def body(t_ref, batch_ref, nct_ref, wt_ref, bt_ref, out_ref):
        gid_col = lax.broadcasted_iota(jnp.int32, (num_graphs, 1), 0)

        def cbody(i, acc):
            row = batch_ref[pl.ds(i, 1), :]
            eq = (row == gid_col).astype(jnp.float32)
            return acc + jnp.sum(eq, axis=1, keepdims=True)

        counts = lax.fori_loop(
            0, nblk, cbody, jnp.zeros((num_graphs, 1), jnp.float32))
        n_idx = jnp.clip(counts.astype(jnp.int32), 0, ncv - 1)
        onehot = (n_idx == lax.broadcasted_iota(
            jnp.int32, (1, ncv), 1)).astype(jnp.float32)
        n_embed = jnp.dot(onehot, nct_ref[...],
                          preferred_element_type=jnp.float32,
                          precision=lax.Precision.HIGHEST)

        freqs = jnp.exp(
            (-math.log(10000.0) / half)
            * lax.broadcasted_iota(jnp.int32, (1, half), 1).astype(jnp.float32))
        args = t_ref[...] * freqs
        temb = jnp.concatenate([jnp.sin(args), jnp.cos(args)], axis=-1)
        t_embed = jnp.dot(temb, wt_ref[...],
                          preferred_element_type=jnp.float32) + bt_ref[...]

        out_ref[:, :tdim] = t_embed
        out_ref[:, tdim:] = n_embed

    return pl.pallas_call(
        body,
        out_shape=jax.ShapeDtypeStruct((num_graphs, tdim + ncd), jnp.float32),
    )(t_col, batch2d, node_count_table, W_t, b_row)


def kernel(a, e, edge_index, t, batch, atom_table, edge_table,
           node_count_table, W_t, b_t):
    del edge_index
    a_embed, e_embed = _sc_gathers(a, e, atom_table, edge_table)
    cond = _tc_cond(
        t.reshape(-1, 1),
        batch.reshape(25, -1),
        node_count_table,
        W_t,
        b_t.reshape(1, -1),
    )
    return a_embed, e_embed, cond
